# baseline (speedup 1.0000x reference)
import functools
import math

import jax
import jax.numpy as jnp
from jax.experimental import pallas as pl
from jax.experimental.pallas import tpu as pltpu

N = 8192
C = 768
L = 2048
DL = 128
G = C + DL
QQ = 512


def _fused_kernel(x_ref, wcat_ref, out_ref, xs_ref, nds_ref, sem):
    j = pl.program_id(0)

    @pl.when(j == 0)
    def _load_x():
        cp = pltpu.make_async_copy(x_ref, xs_ref, sem)
        cp.start()
        cp.wait()

    def _stride2():
        w = xs_ref[pl.ds(2 * L * j, 2 * L), :]
        return w.reshape(L, 2 * C)[:, :C]

    def _stride4():
        return xs_ref[...].reshape(L, 4 * C)[:, :C]

    def _stride1():
        return xs_ref[pl.ds(L * (j - 3), L), :]

    branch = (j >= 2).astype(jnp.int32) + (j >= 3).astype(jnp.int32)
    xb = jax.lax.switch(branch, [_stride2, _stride4, _stride1])

    wcat = wcat_ref[...]
    q16 = jnp.dot(xb, wcat[:, :C],
                  preferred_element_type=jnp.float32).astype(jnp.bfloat16)
    k16 = jnp.dot(xb, wcat[:, C:2 * C],
                  preferred_element_type=jnp.float32).astype(jnp.bfloat16)
    v16 = jnp.dot(xb, wcat[:, 2 * C:],
                  preferred_element_type=jnp.float32).astype(jnp.bfloat16)
    ones = jnp.ones((L, DL), jnp.bfloat16)
    base = jnp.where(j < 3, L * j, 3 * L)
    for qq in range(L // QQ):
        s = jax.lax.dot_general(
            q16[qq * QQ:(qq + 1) * QQ, :], k16, (((1,), (1,)), ((), ())),
            preferred_element_type=jnp.float32)
        e16 = jnp.exp(s).astype(jnp.bfloat16)
        nmq = jnp.dot(e16, v16, preferred_element_type=jnp.float32)
        dnq = jnp.dot(e16, ones, preferred_element_type=jnp.float32)
        rows = pl.ds(base + qq * QQ, QQ)
        nds_ref[rows, :C] = nmq.astype(jnp.bfloat16)
        nds_ref[rows, C:] = dnq.astype(jnp.bfloat16)

    @pl.when(j >= 3)
    def _mix():
        b = j - 3
        for h in range(2):
            R = 256
            a1 = nds_ref[pl.ds(3 * L + 1024 * h, 1024), :]
            a2 = nds_ref[pl.ds(1024 * b + 512 * h, 512), :]
            a3 = nds_ref[pl.ds(2 * L + 512 * b + 256 * h, 256), :]
            A = a1.reshape(R, 4 * G).astype(jnp.float32)
            P = a2.reshape(R, 2 * G).astype(jnp.float32)
            S = a3.astype(jnp.float32)
            z = jnp.zeros((R, G), jnp.float32)
            tot = A + jnp.concatenate(
                [P[:, :G], z, P[:, G:], z], axis=1) + jnp.concatenate(
                [S, z, z, z], axis=1)
            outq = jnp.concatenate(
                [tot[:, g * G:g * G + C] / tot[:, g * G + C:g * G + C + 1]
                 for g in range(4)], axis=1)
            out_ref[pl.ds(1024 * h, 1024), :] = outq.reshape(1024, C)


def kernel(x, Wq, Wk, Wv):
    x16 = x[0].astype(jnp.bfloat16)
    scale = 1.0 / math.sqrt(C)
    Wcat = jnp.concatenate([Wq * scale, Wk, Wv], axis=1).astype(jnp.bfloat16)
    out = pl.pallas_call(
        _fused_kernel,
        grid=(7,),
        in_specs=[
            pl.BlockSpec(memory_space=pl.ANY),
            pl.BlockSpec((C, 3 * C), lambda j: (0, 0)),
        ],
        out_specs=pl.BlockSpec(
            (L, C), lambda j: (jnp.maximum(j - 3, 0), 0)),
        out_shape=jax.ShapeDtypeStruct((N, C), jnp.float32),
        scratch_shapes=[
            pltpu.VMEM((N, C), jnp.bfloat16),
            pltpu.VMEM((4 * L, G), jnp.bfloat16),
            pltpu.SemaphoreType.DMA,
        ],
        compiler_params=pltpu.CompilerParams(
            vmem_limit_bytes=64 * 1024 * 1024),
    )(x16, Wcat)
    return out.reshape(1, N, C)


# speedup vs baseline: 1.5298x; 1.5298x over previous
import functools
import math

import jax
import jax.numpy as jnp
import numpy as np
from jax.experimental import pallas as pl
from jax.experimental.pallas import tpu as pltpu

N = 8192
C = 768
L = 2048
DL = 128
G = C + DL


def _attn_kernel(x_ref, wcat_ref, num_ref, den_ref, xg_ref):
    j = pl.program_id(0)

    @pl.when(j < 4)
    def _deposit():
        blk = x_ref[...]
        xg_ref[pl.ds(1024 * j, 1024), :] = blk.reshape(1024, 2 * C)[:, :C]
        xg_ref[pl.ds(2 * L + 512 * j, 512), :] = (
            blk.reshape(512, 4 * C)[:, :C])

    def _from_x():
        return x_ref[...]

    def _from_g():
        return xg_ref[pl.ds(L * (j - 4), L), :]

    xb = jax.lax.cond(j < 4, _from_x, _from_g)

    wcat = wcat_ref[...]
    q16 = jnp.dot(xb, wcat[:, :C],
                  preferred_element_type=jnp.float32).astype(jnp.bfloat16)
    k16 = jnp.dot(xb, wcat[:, C:2 * C],
                  preferred_element_type=jnp.float32).astype(jnp.bfloat16)
    v16 = jnp.dot(xb, wcat[:, 2 * C:],
                  preferred_element_type=jnp.float32).astype(jnp.bfloat16)
    H = L // 2
    e_halves = []
    for h in range(2):
        s = jax.lax.dot_general(
            q16[h * H:(h + 1) * H, :], k16, (((1,), (1,)), ((), ())),
            preferred_element_type=jnp.float32)
        e_halves.append(jnp.exp(s).astype(jnp.bfloat16))
    vcat = jnp.concatenate(
        [v16, jnp.ones((L, DL), jnp.bfloat16)], axis=1)
    for h in range(2):
        nd = jnp.dot(e_halves[h], vcat, preferred_element_type=jnp.float32)
        nd16 = nd.astype(jnp.bfloat16)
        num_ref[h * H:(h + 1) * H, :] = nd16[:, :C]
        den_ref[h * H:(h + 1) * H, :] = nd16[:, C:]


def _run_segs(x16, Wcat):
    return pl.pallas_call(
        _attn_kernel,
        grid=(7,),
        in_specs=[
            pl.BlockSpec((L, C), lambda j: (jnp.minimum(j, 3), 0)),
            pl.BlockSpec((C, 3 * C), lambda j: (0, 0)),
        ],
        out_specs=[
            pl.BlockSpec((L, C), lambda j: (j, 0)),
            pl.BlockSpec((L, DL), lambda j: (j, 0)),
        ],
        out_shape=[
            jax.ShapeDtypeStruct((7 * L, C), jnp.bfloat16),
            jax.ShapeDtypeStruct((7 * L, DL), jnp.bfloat16),
        ],
        scratch_shapes=[
            pltpu.VMEM((3 * L, C), jnp.bfloat16),
        ],
    )(x16, Wcat)


def _mix_kernel(u_ref, n1_ref, n2_ref, n3_ref, d1_ref, d2_ref, d3_ref,
                out_ref):
    nd23 = jnp.concatenate([
        jnp.concatenate([n2_ref[...], d2_ref[...]], axis=1),
        jnp.concatenate([n3_ref[...], d3_ref[...]], axis=1),
    ], axis=0)
    mm = jnp.dot(u_ref[...], nd23, preferred_element_type=jnp.float32)
    ntot = n1_ref[...].astype(jnp.float32) + mm[:, :C]
    dtot = d1_ref[...].astype(jnp.float32) + mm[:, C:]
    out_ref[...] = ntot / dtot[:, 0:1]


def _mix(num, den, U):
    return pl.pallas_call(
        _mix_kernel,
        grid=(4,),
        in_specs=[
            pl.BlockSpec((L, 3 * L // 4), lambda j: (0, 0)),
            pl.BlockSpec((L, C), lambda j: (j, 0)),
            pl.BlockSpec((L // 2, C), lambda j: (8 + j, 0)),
            pl.BlockSpec((L // 4, C), lambda j: (24 + j, 0)),
            pl.BlockSpec((L, DL), lambda j: (j, 0)),
            pl.BlockSpec((L // 2, DL), lambda j: (8 + j, 0)),
            pl.BlockSpec((L // 4, DL), lambda j: (24 + j, 0)),
        ],
        out_specs=pl.BlockSpec((L, C), lambda j: (j, 0)),
        out_shape=jax.ShapeDtypeStruct((N, C), jnp.float32),
    )(U, num, num, num, den, den, den)


def _interleave_matrix():
    u = np.zeros((L, 3 * L // 4), np.float32)
    u[2 * np.arange(L // 2), np.arange(L // 2)] = 1.0
    u[4 * np.arange(L // 4), L // 2 + np.arange(L // 4)] = 1.0
    return u


_U_NP = _interleave_matrix()


def kernel(x, Wq, Wk, Wv):
    x16 = x[0].astype(jnp.bfloat16)
    scale = 1.0 / math.sqrt(C)
    Wcat = jnp.concatenate([Wq * scale, Wk, Wv], axis=1).astype(jnp.bfloat16)
    U = jnp.asarray(_U_NP, dtype=jnp.bfloat16)
    num, den = _run_segs(x16, Wcat)
    out = _mix(num, den, U)
    return out.reshape(1, N, C)


# speedup vs baseline: 1.5314x; 1.0010x over previous
import functools
import math

import jax
import jax.numpy as jnp
import numpy as np
from jax.experimental import pallas as pl
from jax.experimental.pallas import tpu as pltpu

N = 8192
C = 768
L = 2048
DL = 128
G = C + DL


def _attn_kernel(x_ref, wcat_ref, nd_ref, xg_ref):
    j = pl.program_id(0)

    @pl.when(j < 4)
    def _deposit():
        blk = x_ref[...]
        xg_ref[pl.ds(1024 * j, 1024), :] = blk.reshape(1024, 2 * C)[:, :C]
        xg_ref[pl.ds(2 * L + 512 * j, 512), :] = (
            blk.reshape(512, 4 * C)[:, :C])

    def _from_x():
        return x_ref[...]

    def _from_g():
        return xg_ref[pl.ds(L * (j - 4), L), :]

    xb = jax.lax.cond(j < 4, _from_x, _from_g)

    wcat = wcat_ref[...]
    q16 = jnp.dot(xb, wcat[:, :C],
                  preferred_element_type=jnp.float32).astype(jnp.bfloat16)
    k16 = jnp.dot(xb, wcat[:, C:2 * C],
                  preferred_element_type=jnp.float32).astype(jnp.bfloat16)
    v16 = jnp.dot(xb, wcat[:, 2 * C:],
                  preferred_element_type=jnp.float32).astype(jnp.bfloat16)
    H = L // 2
    e_halves = []
    for h in range(2):
        s = jax.lax.dot_general(
            q16[h * H:(h + 1) * H, :], k16, (((1,), (1,)), ((), ())),
            preferred_element_type=jnp.float32)
        e_halves.append(jnp.exp(s).astype(jnp.bfloat16))
    vcat = jnp.concatenate(
        [v16, jnp.ones((L, DL), jnp.bfloat16)], axis=1)
    for h in range(2):
        nd = jnp.dot(e_halves[h], vcat, preferred_element_type=jnp.float32)
        nd_ref[h * H:(h + 1) * H, :] = nd.astype(jnp.bfloat16)


def _run_segs(x16, Wcat):
    return pl.pallas_call(
        _attn_kernel,
        grid=(7,),
        in_specs=[
            pl.BlockSpec((L, C), lambda j: (jnp.minimum(j, 3), 0)),
            pl.BlockSpec((C, 3 * C), lambda j: (0, 0)),
        ],
        out_specs=[
            pl.BlockSpec((L, G), lambda j: (j, 0)),
        ],
        out_shape=[
            jax.ShapeDtypeStruct((7 * L, G), jnp.bfloat16),
        ],
        scratch_shapes=[
            pltpu.VMEM((3 * L, C), jnp.bfloat16),
        ],
    )(x16, Wcat)


def _mix_kernel(u_ref, nd1_ref, nd2_ref, nd3_ref, out_ref):
    nd23 = jnp.concatenate([nd2_ref[...], nd3_ref[...]], axis=0)
    mm = jnp.dot(u_ref[...], nd23, preferred_element_type=jnp.float32)
    tot = nd1_ref[...].astype(jnp.float32) + mm
    out_ref[...] = tot[:, :C] / tot[:, C:C + 1]


def _mix(nd, U):
    return pl.pallas_call(
        _mix_kernel,
        grid=(4,),
        in_specs=[
            pl.BlockSpec((L, 3 * L // 4), lambda j: (0, 0)),
            pl.BlockSpec((L, G), lambda j: (j, 0)),
            pl.BlockSpec((L // 2, G), lambda j: (8 + j, 0)),
            pl.BlockSpec((L // 4, G), lambda j: (24 + j, 0)),
        ],
        out_specs=pl.BlockSpec((L, C), lambda j: (j, 0)),
        out_shape=jax.ShapeDtypeStruct((N, C), jnp.float32),
    )(U, nd, nd, nd)


def _interleave_matrix():
    u = np.zeros((L, 3 * L // 4), np.float32)
    u[2 * np.arange(L // 2), np.arange(L // 2)] = 1.0
    u[4 * np.arange(L // 4), L // 2 + np.arange(L // 4)] = 1.0
    return u


_U_NP = _interleave_matrix()


def kernel(x, Wq, Wk, Wv):
    x16 = x[0].astype(jnp.bfloat16)
    scale = 1.0 / math.sqrt(C)
    Wcat = jnp.concatenate([Wq * scale, Wk, Wv], axis=1).astype(jnp.bfloat16)
    U = jnp.asarray(_U_NP, dtype=jnp.bfloat16)
    (nd,) = _run_segs(x16, Wcat)
    out = _mix(nd, U)
    return out.reshape(1, N, C)
